# Initial kernel scaffold; baseline (speedup 1.0000x reference)
import jax
import jax.numpy as jnp
from jax.experimental import pallas as pl


def kernel(indices, batch_ids, values, weight):
    raise NotImplementedError("write your pallas kernel here")


# speedup vs baseline: 2.4690x; 2.4690x over previous
import functools

import jax
import jax.numpy as jnp
from jax import lax
from jax.experimental import pallas as pl
from jax.experimental.pallas import tpu as pltpu
from jax.experimental.pallas import tpu_sc as plsc

NC = 2
NS = 16
L = 16
NW = NC * NS
SUB = 128
D = 16
B = 4096


def _iota16():
    return lax.broadcasted_iota(jnp.int32, (L,), 0)


VOCAB = 1000000
NBLK = (VOCAB + 127) // 128
NFULL = VOCAB // 128
BLK_PER_TILE = (NBLK + NW - 1) // NW


@functools.lru_cache(maxsize=None)
def _phase0():
    mesh = plsc.VectorSubcoreMesh(core_axis_name="c", subcore_axis_name="s")
    WV = 1024
    NWIDE = (VOCAB // 128 // 8)
    KMAX = (NWIDE + NW - 1) // NW
    NREM = NFULL - NWIDE * 8

    @functools.partial(
        pl.kernel,
        out_type=jax.ShapeDtypeStruct((VOCAB * D,), jnp.float32),
        mesh=mesh,
        scratch_types=[
            pltpu.VMEM((2 * D, WV + 1), jnp.float32),
            pltpu.VMEM((2 * WV * D,), jnp.float32),
            pltpu.SemaphoreType.DMA((2,)),
            pltpu.SemaphoreType.DMA((2,)),
        ],
        compiler_params=pltpu.CompilerParams(
            needs_layout_passes=False, use_tc_tiling_on_sc=True),
    )
    def k0(wt, wtail, out, tbuf, obuf, insem, outsem):
        cid = lax.axis_index("c")
        sid = lax.axis_index("s")
        wid = sid * NC + cid
        iota = _iota16()

        def in_copies(g, b):
            lo = pltpu.make_async_copy(
                wt.at[pl.ds(0, 8), pl.ds(g * WV, WV)],
                tbuf.at[pl.ds(b * D, 8), pl.ds(0, WV)], insem.at[b])
            hi = pltpu.make_async_copy(
                wt.at[pl.ds(8, 8), pl.ds(g * WV, WV)],
                tbuf.at[pl.ds(b * D + 8, 8), pl.ds(0, WV)], insem.at[b])
            return lo, hi

        def out_copy(g, b):
            return pltpu.make_async_copy(
                obuf.at[pl.ds(b * WV * D, WV * D)],
                out.at[pl.ds(g * WV * D, WV * D)], outsem.at[b])

        def transpose(b, nrows):
            rowsel = iota + b * D
            obase = b * WV * D
            ones = jnp.ones((L,), jnp.int32)

            def body(v, colsel):
                col = plsc.load_gather(tbuf, [rowsel, colsel])
                obuf[pl.ds(obase + v * D, D)] = col
                return colsel + ones
            plsc.parallel_loop(0, nrows, unroll=16,
                               carry=jnp.zeros((L,), jnp.int32))(body)

        for b in range(2):
            lo, hi = in_copies(wid + b * NW, b)
            lo.start()
            hi.start()

        def slot(k, _):
            b = lax.rem(k, 2)
            g = wid + k * NW

            @pl.when(g < NWIDE)
            def _run():
                lo, hi = in_copies(g, b)
                lo.wait()
                hi.wait()

                @pl.when(k >= 2)
                def _drain():
                    out_copy(g - 2 * NW, b).wait()
                transpose(b, WV)
                out_copy(g, b).start()
                g2 = g + 2 * NW

                @pl.when(g2 < NWIDE)
                def _prefetch():
                    lo2, hi2 = in_copies(g2, b)
                    lo2.start()
                    hi2.start()
            return 0
        lax.fori_loop(0, KMAX, slot, 0)

        for b in range(2):
            last_k = jnp.where((wid + (KMAX - 1) * NW < NWIDE),
                               KMAX - 1, KMAX - 2)
            kb = jnp.where(lax.rem(last_k, 2) == b, last_k, last_k - 1)
            out_copy(wid + kb * NW, b).wait()

        @pl.when(wid < NREM)
        def _rem():
            c = NWIDE * 8 + wid
            pltpu.sync_copy(wt.at[pl.ds(0, 8), pl.ds(c * 128, 128)],
                            tbuf.at[pl.ds(0, 8), pl.ds(0, 128)])
            pltpu.sync_copy(wt.at[pl.ds(8, 8), pl.ds(c * 128, 128)],
                            tbuf.at[pl.ds(8, 8), pl.ds(0, 128)])
            transpose(0, 128)
            pltpu.sync_copy(obuf.at[pl.ds(0, 128 * D)],
                            out.at[pl.ds(c * 128 * D, 128 * D)])

        ntail = VOCAB - NFULL * 128

        @pl.when(wid == NW - 1)
        def _tail():
            pltpu.sync_copy(wtail.at[pl.ds(0, 8)],
                            tbuf.at[pl.ds(0, 8), pl.ds(0, 128)])
            pltpu.sync_copy(wtail.at[pl.ds(8, 8)],
                            tbuf.at[pl.ds(8, 8), pl.ds(0, 128)])
            transpose(0, ntail)
            pltpu.sync_copy(obuf.at[pl.ds(0, ntail * D)],
                            out.at[pl.ds(NFULL * 128 * D, ntail * D)])

    return k0


@functools.lru_cache(maxsize=None)
def _phase1(nsub):
    rows_per_tile = B // NS
    mesh = plsc.VectorSubcoreMesh(core_axis_name="c", subcore_axis_name="s")

    @functools.partial(
        pl.kernel,
        out_type=(
            jax.ShapeDtypeStruct((NC, B, D), jnp.float32),
            jax.ShapeDtypeStruct((NC, B), jnp.float32),
        ),
        mesh=mesh,
        scratch_types=[
            pltpu.VMEM((nsub, SUB), jnp.int32),
            pltpu.VMEM((nsub, SUB), jnp.int32),
            pltpu.VMEM((nsub, SUB), jnp.float32),
            pltpu.VMEM((2 * SUB, D), jnp.float32),
            pltpu.VMEM((2 * SUB, D), jnp.float32),
            pltpu.VMEM((2 * SUB, D), jnp.float32),
            pltpu.VMEM((rows_per_tile, D), jnp.float32),
            pltpu.VMEM((rows_per_tile,), jnp.float32),
            pltpu.VMEM_SHARED((B, D), jnp.float32),
            pltpu.VMEM_SHARED((B, D), jnp.float32),
            pltpu.SemaphoreType.DMA((2,)),
            pltpu.SemaphoreType.DMA((2,)),
        ],
        compiler_params=pltpu.CompilerParams(needs_layout_passes=False, use_tc_tiling_on_sc=False),
    )
    def k1(weight, idx3, bid3, val3, acc_out, self_out,
           idx_v, bid_v, val_v, rows_v, scaled_v, sq_v, pub_v, self_v,
           acc_sh, sq_sh, gsem, ssem):
        cid = lax.axis_index("c")
        sid = lax.axis_index("s")
        wid = sid * NC + cid
        rbase = _iota16()
        zero16 = jnp.zeros((L,), jnp.float32)

        def zbody(i, _):
            pub_v[i] = zero16
            return 0
        lax.fori_loop(0, rows_per_tile, zbody, 0)
        tile_rows = pl.ds(sid * rows_per_tile, rows_per_tile)
        pltpu.sync_copy(pub_v, acc_sh.at[tile_rows])
        pltpu.sync_copy(pub_v, sq_sh.at[tile_rows])
        plsc.subcore_barrier()

        pltpu.sync_copy(idx3.at[wid], idx_v)
        pltpu.sync_copy(bid3.at[wid], bid_v)
        pltpu.sync_copy(val3.at[wid], val_v)

        def gat(j, b):
            return pltpu.make_async_copy(
                weight.at[idx_v.at[j]], rows_v.at[pl.ds(b * SUB, SUB)],
                gsem.at[b])

        for b in range(2):
            gat(b, b).start()

        def chunk(j, _):
            b = lax.rem(j, 2)
            gat(j, b).wait()

            @pl.when(j >= 2)
            def _drain():
                pltpu.make_async_copy(
                    scaled_v.at[pl.ds(b * SUB, SUB)],
                    acc_sh.at[bid_v.at[j - 2]], ssem.at[b]).wait()
                pltpu.make_async_copy(
                    sq_v.at[pl.ds(b * SUB, SUB)],
                    sq_sh.at[bid_v.at[j - 2]], ssem.at[b]).wait()

            def grp(g):
                rid = rbase + g * L + b * SUB
                vals16 = val_v[j, pl.ds(g * L, L)]
                for d in range(D):
                    dd = jnp.full((L,), d, jnp.int32)
                    col = plsc.load_gather(rows_v, [rid, dd])
                    s = col * vals16
                    plsc.store_scatter(scaled_v, [rid, dd], s)
                    plsc.store_scatter(sq_v, [rid, dd], s * s)
            plsc.parallel_loop(0, SUB // L, unroll=2)(grp)

            pltpu.async_copy(scaled_v.at[pl.ds(b * SUB, SUB)],
                             acc_sh.at[bid_v.at[j]], ssem.at[b], add=True)
            pltpu.async_copy(sq_v.at[pl.ds(b * SUB, SUB)],
                             sq_sh.at[bid_v.at[j]], ssem.at[b], add=True)

            @pl.when(j + 2 < nsub)
            def _prefetch():
                gat(j + 2, b).start()
            return 0
        lax.fori_loop(0, nsub, chunk, 0)

        for b in range(2):
            jl = nsub - 2 + b
            pltpu.make_async_copy(
                scaled_v.at[pl.ds(b * SUB, SUB)],
                acc_sh.at[bid_v.at[jl]], ssem.at[b]).wait()
            pltpu.make_async_copy(
                sq_v.at[pl.ds(b * SUB, SUB)],
                sq_sh.at[bid_v.at[jl]], ssem.at[b]).wait()

        plsc.subcore_barrier()

        pltpu.sync_copy(acc_sh.at[tile_rows], pub_v)
        pltpu.sync_copy(pub_v, acc_out.at[cid, tile_rows])
        pltpu.sync_copy(sq_sh.at[tile_rows], pub_v)
        def redgrp(g):
            rid = rbase + g * L
            tot = zero16
            for d in range(D):
                tot = tot + plsc.load_gather(pub_v, [rid, jnp.full((L,), d, jnp.int32)])
            self_v[pl.ds(g * L, L)] = tot
        plsc.parallel_loop(0, rows_per_tile // L, unroll=2)(redgrp)
        pltpu.sync_copy(self_v, self_out.at[cid, tile_rows])

    return k1


@functools.lru_cache(maxsize=None)
def _phase2():
    bw = B // NW
    mesh = plsc.VectorSubcoreMesh(core_axis_name="c", subcore_axis_name="s")

    @functools.partial(
        pl.kernel,
        out_type=jax.ShapeDtypeStruct((B,), jnp.float32),
        mesh=mesh,
        scratch_types=[
            pltpu.VMEM((bw, D), jnp.float32),
            pltpu.VMEM((bw, D), jnp.float32),
            pltpu.VMEM((bw,), jnp.float32),
            pltpu.VMEM((bw,), jnp.float32),
            pltpu.VMEM((bw,), jnp.float32),
        ],
        compiler_params=pltpu.CompilerParams(needs_layout_passes=False, use_tc_tiling_on_sc=False),
    )
    def k2(acc_part, self_part, out, a0, a1, s0, s1, o):
        cid = lax.axis_index("c")
        sid = lax.axis_index("s")
        wid = sid * NC + cid
        bsl = pl.ds(wid * bw, bw)
        pltpu.sync_copy(acc_part.at[0, bsl], a0)
        pltpu.sync_copy(acc_part.at[1, bsl], a1)
        pltpu.sync_copy(self_part.at[0, bsl], s0)
        pltpu.sync_copy(self_part.at[1, bsl], s1)
        rbase = _iota16()

        def outgrp(g):
            rid = rbase + g * L
            acc16 = -(s0[pl.ds(g * L, L)] + s1[pl.ds(g * L, L)])
            for d in range(D):
                dd = jnp.full((L,), d, jnp.int32)
                t = plsc.load_gather(a0, [rid, dd]) + plsc.load_gather(a1, [rid, dd])
                acc16 = acc16 + t * t
            o[pl.ds(g * L, L)] = acc16
        plsc.parallel_loop(0, bw // L, unroll=2)(outgrp)
        pltpu.sync_copy(o, out.at[bsl])

    return k2


def kernel(indices, batch_ids, values, weight):
    nnz = indices.shape[0]
    assert nnz % (NW * SUB) == 0
    nsub = nnz // (NW * SUB)
    idx3 = indices.reshape(NW, nsub, SUB)
    bid3 = batch_ids.reshape(NW, nsub, SUB)
    val3 = values.reshape(NW, nsub, SUB)
    wt = weight.T
    wtail = jnp.pad(wt[:, NFULL * 128:], ((0, 0), (0, 128 - (VOCAB - NFULL * 128))))
    w_lin = _phase0()(wt, wtail).reshape(VOCAB, D)
    acc_part, self_part = _phase1(nsub)(w_lin, idx3, bid3, val3)
    out = _phase2()(acc_part, self_part)
    return out.reshape(-1, 1)
